# baseline (speedup 1.0000x reference)
import functools

import jax
import jax.numpy as jnp
from jax import lax
from jax.experimental import pallas as pl
from jax.experimental.pallas import tpu as pltpu
from jax.experimental.pallas import tpu_sc as plsc

_BS = 32
_NP = 8732
_NPP = 8736
_NCH = _NPP // 16
_NO = 16
_NC = 81


def _sc_match_body(obj_hbm, pxy_hbm, out_hbm,
                   objv, px1v, py1v, px2v, py2v, ovb, ojb, selb):
    i = lax.axis_index("s") * 2 + lax.axis_index("c")
    pltpu.sync_copy(obj_hbm.at[i], objv)
    pltpu.sync_copy(pxy_hbm.at[0], px1v)
    pltpu.sync_copy(pxy_hbm.at[1], py1v)
    pltpu.sync_copy(pxy_hbm.at[2], px2v)
    pltpu.sync_copy(pxy_hbm.at[3], py2v)

    lane = lax.broadcasted_iota(jnp.int32, (16,), 0)
    ox1v = objv[0]
    oy1v = objv[1]
    ox2v = objv[2]
    oy2v = objv[3]
    ox1s = [ox1v[j] for j in range(_NO)]
    oy1s = [oy1v[j] for j in range(_NO)]
    ox2s = [ox2v[j] for j in range(_NO)]
    oy2s = [oy2v[j] for j in range(_NO)]

    def chunk(c, carry):
        mxs, ids = carry
        base = c * 16
        p1 = px1v[pl.ds(base, 16)]
        q1 = py1v[pl.ds(base, 16)]
        p2 = px2v[pl.ds(base, 16)]
        q2 = py2v[pl.ds(base, 16)]
        area_p = (p2 - p1) * (q2 - q1)
        bov = jnp.full((16,), -1.0, jnp.float32)
        boj = jnp.zeros((16,), jnp.int32)
        pb_idx = base + lane
        new_mxs = []
        new_ids = []
        for j in range(_NO):
            ox1 = ox1s[j]
            oy1 = oy1s[j]
            ox2 = ox2s[j]
            oy2 = oy2s[j]
            wdt = jnp.maximum(jnp.minimum(p2, ox2) - jnp.maximum(p1, ox1),
                              0.0)
            hgt = jnp.maximum(jnp.minimum(q2, oy2) - jnp.maximum(q1, oy1),
                              0.0)
            inter = wdt * hgt
            area_o = (ox2 - ox1) * (oy2 - oy1)
            iou = inter / (area_o + area_p - inter)
            upd = iou > bov
            boj = jnp.where(upd, j, boj)
            bov = jnp.where(upd, iou, bov)
            upd2 = iou > mxs[j]
            new_ids.append(jnp.where(upd2, pb_idx, ids[j]))
            new_mxs.append(jnp.where(upd2, iou, mxs[j]))
        ovb[pl.ds(base, 16)] = bov
        ojb[pl.ds(base, 16)] = boj
        return tuple(new_mxs), tuple(new_ids)

    mx0 = tuple(jnp.full((16,), -1.0, jnp.float32) for _ in range(_NO))
    id0 = tuple(jnp.zeros((16,), jnp.int32) for _ in range(_NO))
    mxs, ids = lax.fori_loop(0, _NCH, chunk, (mx0, id0))

    dn = lax.GatherDimensionNumbers(offset_dims=(),
                                    collapsed_slice_dims=(0,),
                                    start_index_map=(0,))

    def perm(x, pm):
        return lax.gather(x, pm[:, None], dn, (1,),
                          mode=lax.GatherScatterMode.PROMISE_IN_BOUNDS)

    mxs_l = list(mxs)
    ids_l = list(ids)
    for r in (8, 4, 2, 1):
        pm = (lane + r) & 15
        for j in range(_NO):
            my = perm(mxs_l[j], pm)
            iy = perm(ids_l[j], pm)
            take = (my > mxs_l[j]) | ((my == mxs_l[j]) & (iy < ids_l[j]))
            ids_l[j] = jnp.where(take, iy, ids_l[j])
            mxs_l[j] = jnp.maximum(mxs_l[j], my)

    def selchunk(c, carry):
        base = c * 16
        ovv = ovb[pl.ds(base, 16)]
        ojv = ojb[pl.ds(base, 16)]
        sel = ojv + jnp.where(ovv < 0.5, 16, 0)
        pb_idx = base + lane
        for j in range(_NO):
            sel = jnp.where(pb_idx == ids_l[j], j, sel)
        selb[pl.ds(base, 16)] = sel
        return carry

    lax.fori_loop(0, _NCH, selchunk, 0)
    pltpu.sync_copy(selb, out_hbm.at[i])


def _sc_match(obj4, pxy):
    f = pl.kernel(
        _sc_match_body,
        mesh=plsc.VectorSubcoreMesh(core_axis_name="c", subcore_axis_name="s"),
        out_type=jax.ShapeDtypeStruct((_BS, _NPP), jnp.int32),
        scratch_types=[
            pltpu.VMEM((4, 16), jnp.float32),
            pltpu.VMEM((_NPP,), jnp.float32),
            pltpu.VMEM((_NPP,), jnp.float32),
            pltpu.VMEM((_NPP,), jnp.float32),
            pltpu.VMEM((_NPP,), jnp.float32),
            pltpu.VMEM((_NPP,), jnp.float32),
            pltpu.VMEM((_NPP,), jnp.int32),
            pltpu.VMEM((_NPP,), jnp.int32),
        ],
    )
    return f(obj4, pxy)


def _dense_body(tc_ref, sc_ref, lse_ref, cand_ref):
    s = sc_ref[0]
    m = jnp.max(s)
    e = jnp.exp(s - m)
    ones = jnp.ones((1, _NC), jnp.float32)
    se_t = lax.dot_general(ones, e, (((1,), (1,)), ((), ())),
                           preferred_element_type=jnp.float32)
    lse_ref[0] = jnp.log(se_t) + m

    tcc = jnp.concatenate([tc_ref[0], jnp.zeros((1, 1), jnp.int32)],
                          axis=0)
    cid81 = lax.broadcasted_iota(jnp.int32, (_NO + 1, _NC), 1)
    tco = (cid81 == tcc).astype(jnp.float32)
    cand_ref[0] = lax.dot_general(tco, s, (((1,), (1,)), ((), ())),
                                  preferred_element_type=jnp.float32)


def _image_body(sel_ref, tb_ref, tc_ref, pbc_ref, pb_ref, lse_ref, cand_ref,
                cls_ref, stats_ref):
    selp = sel_ref[0][:, :_NP]
    neg = selp >= 16
    obj = jnp.where(neg, selp - 16, selp)

    jidx = lax.broadcasted_iota(jnp.int32, (_NO, _NP), 0)
    onehotf = (obj == jidx).astype(jnp.float32)
    tb = tb_ref[0]
    tb5 = jnp.concatenate([tb, tc_ref[0].astype(jnp.float32)], axis=1)
    gath = lax.dot_general(tb5, onehotf, (((0,), (0,)), ((), ())),
                           preferred_element_type=jnp.float32)
    gx1 = gath[0:1, :]
    gy1 = gath[1:2, :]
    gx2 = gath[2:3, :]
    gy2 = gath[3:4, :]
    labels = jnp.where(neg, 0, gath[4:5, :].astype(jnp.int32))

    cx = (gx1 + gx2) * 0.5
    cy = (gy1 + gy2) * 0.5
    bw = gx2 - gx1
    bh = gy2 - gy1
    pcx = pbc_ref[0:1, :]
    pcy = pbc_ref[1:2, :]
    pw = pbc_ref[2:3, :]
    ph = pbc_ref[3:4, :]
    gcx = (cx - pcx) / (pw * 0.1)
    gcy = (cy - pcy) / (ph * 0.1)
    gw = jnp.log(bw / pw) * 5.0
    gh = jnp.log(bh / ph) * 5.0

    posf = (labels != 0).astype(jnp.float32)
    pb = pb_ref[0]
    locsum = (jnp.sum(jnp.abs(pb[0:1, :] - gcx) * posf)
              + jnp.sum(jnp.abs(pb[1:2, :] - gcy) * posf)
              + jnp.sum(jnp.abs(pb[2:3, :] - gw) * posf)
              + jnp.sum(jnp.abs(pb[3:4, :] - gh) * posf))
    npos = jnp.sum(posf)

    lse_t = lse_ref[0]
    cand = cand_ref[0]
    k = jnp.where(neg, _NO, obj)
    rid = lax.broadcasted_iota(jnp.int32, (_NO + 1, _NP), 0)
    csel = jnp.where(rid == k, cand, 0.0)
    ones17 = jnp.ones((1, _NO + 1), jnp.float32)
    s_at_t = lax.dot_general(ones17, csel, (((1,), (0,)), ((), ())),
                             preferred_element_type=jnp.float32)
    cls_t = lse_t - s_at_t
    cls_ref[0] = cls_t

    stats_ref[0, 0:1, :] = jnp.full((1, 128), locsum, jnp.float32)
    stats_ref[0, 1:2, :] = jnp.full((1, 128), npos, jnp.float32)
    stats_ref[0, 2:3, :] = jnp.full((1, 128), jnp.sum(cls_t * posf),
                                    jnp.float32)
    stats_ref[0, 3:4, :] = jnp.zeros((1, 128), jnp.float32)


def _combine_body(cls_ref, stats_ref, out_ref):
    cls = cls_ref[...]
    stats = stats_ref[...]
    locsum = jnp.sum(stats[:, 0:1, 0:1])
    npos = jnp.sum(stats[:, 1:2, 0:1])
    clspos = jnp.sum(stats[:, 2:3, 0:1])

    k = jnp.minimum((3.0 * npos).astype(jnp.int32), _NP)
    bits = lax.bitcast_convert_type(cls, jnp.int32)

    def step(_, carry):
        lo, hi = carry
        mid = lo + ((hi - lo) >> 1)
        cnt = jnp.sum((bits >= mid).astype(jnp.int32), axis=2,
                      keepdims=True)
        ge = cnt >= k
        return jnp.where(ge, mid, lo), jnp.where(ge, hi, mid)

    lo0 = jnp.zeros((_BS, 1, 1), jnp.int32)
    hi0 = jnp.full((_BS, 1, 1), 0x7F800000, jnp.int32)
    lo, _ = lax.fori_loop(0, 31, step, (lo0, hi0))
    tval = lax.bitcast_convert_type(lo, jnp.float32)
    gt = bits > lo
    cnt_gt = jnp.sum(gt.astype(jnp.float32), axis=2, keepdims=True)
    sum_gt = jnp.sum(jnp.where(gt, cls, 0.0), axis=2, keepdims=True)
    topk = jnp.sum(sum_gt + (k.astype(jnp.float32) - cnt_gt) * tval)

    loss = locsum / (npos * 4.0) + (clspos + topk) / npos
    out_ref[...] = jnp.full((1, 1), loss, jnp.float32)


@jax.jit
def kernel(pred_boxes, pred_scores, true_boxes, true_classes, pboxes):
    f32 = jnp.float32
    pbc_t = pboxes.T
    pbx_t = jnp.concatenate([pbc_t[:2] - pbc_t[2:] / 2.0,
                             pbc_t[:2] + pbc_t[2:] / 2.0], axis=0)
    pad = jnp.full((4, _NPP - _NP), 2.0, f32)
    pxy = jnp.concatenate([pbx_t, pad], axis=1)
    obj4 = jnp.transpose(true_boxes, (0, 2, 1))
    tc3 = true_classes.reshape(_BS, _NO, 1).astype(jnp.int32)
    pb_t = jnp.transpose(pred_boxes, (0, 2, 1))

    sel = _sc_match(obj4, pxy)
    sel3 = sel.reshape(_BS, 1, _NPP)

    lse, cand = pl.pallas_call(
        _dense_body,
        grid=(_BS,),
        in_specs=[
            pl.BlockSpec((1, _NO, 1), lambda i: (i, 0, 0)),
            pl.BlockSpec((1, _NP, _NC), lambda i: (i, 0, 0)),
        ],
        out_specs=[
            pl.BlockSpec((1, 1, _NP), lambda i: (i, 0, 0)),
            pl.BlockSpec((1, _NO + 1, _NP), lambda i: (i, 0, 0)),
        ],
        out_shape=[
            jax.ShapeDtypeStruct((_BS, 1, _NP), f32),
            jax.ShapeDtypeStruct((_BS, _NO + 1, _NP), f32),
        ],
        compiler_params=pltpu.CompilerParams(
            vmem_limit_bytes=100 * 1024 * 1024),
    )(tc3, pred_scores)

    cls_all, stats = pl.pallas_call(
        _image_body,
        grid=(_BS,),
        in_specs=[
            pl.BlockSpec((1, 1, _NPP), lambda i: (i, 0, 0)),
            pl.BlockSpec((1, _NO, 4), lambda i: (i, 0, 0)),
            pl.BlockSpec((1, _NO, 1), lambda i: (i, 0, 0)),
            pl.BlockSpec((4, _NP), lambda i: (0, 0)),
            pl.BlockSpec((1, 4, _NP), lambda i: (i, 0, 0)),
            pl.BlockSpec((1, 1, _NP), lambda i: (i, 0, 0)),
            pl.BlockSpec((1, _NO + 1, _NP), lambda i: (i, 0, 0)),
        ],
        out_specs=[
            pl.BlockSpec((1, 1, _NP), lambda i: (i, 0, 0)),
            pl.BlockSpec((1, 4, 128), lambda i: (i, 0, 0)),
        ],
        out_shape=[
            jax.ShapeDtypeStruct((_BS, 1, _NP), f32),
            jax.ShapeDtypeStruct((_BS, 4, 128), f32),
        ],
        compiler_params=pltpu.CompilerParams(
            vmem_limit_bytes=100 * 1024 * 1024),
    )(sel3, true_boxes, tc3, pbc_t, pb_t, lse, cand)

    out = pl.pallas_call(
        _combine_body,
        out_shape=jax.ShapeDtypeStruct((1, 1), f32),
    )(cls_all, stats)
    return out[0, 0]


# speedup vs baseline: 1.0784x; 1.0784x over previous
import jax
import jax.numpy as jnp
from jax import lax
from jax.experimental import pallas as pl
from jax.experimental.pallas import tpu as pltpu
from jax.experimental.pallas import tpu_sc as plsc

_BS = 32
_NP = 8732
_NPP = 8736
_NCH = _NPP // 16
_NO = 16
_NC = 81


def _sc_match_body(obj_hbm, pxy_hbm, out_hbm,
                   objv, px1v, py1v, px2v, py2v, ovb, ojb, selb):
    i = lax.axis_index("s") * 2 + lax.axis_index("c")
    pltpu.sync_copy(obj_hbm.at[i], objv)
    pltpu.sync_copy(pxy_hbm.at[0], px1v)
    pltpu.sync_copy(pxy_hbm.at[1], py1v)
    pltpu.sync_copy(pxy_hbm.at[2], px2v)
    pltpu.sync_copy(pxy_hbm.at[3], py2v)

    lane = lax.broadcasted_iota(jnp.int32, (16,), 0)
    ox1v = objv[0]
    oy1v = objv[1]
    ox2v = objv[2]
    oy2v = objv[3]
    ox1s = [ox1v[j] for j in range(_NO)]
    oy1s = [oy1v[j] for j in range(_NO)]
    ox2s = [ox2v[j] for j in range(_NO)]
    oy2s = [oy2v[j] for j in range(_NO)]

    def chunk(c, carry):
        mxs, ids = carry
        base = c * 16
        p1 = px1v[pl.ds(base, 16)]
        q1 = py1v[pl.ds(base, 16)]
        p2 = px2v[pl.ds(base, 16)]
        q2 = py2v[pl.ds(base, 16)]
        area_p = (p2 - p1) * (q2 - q1)
        bov = jnp.full((16,), -1.0, jnp.float32)
        boj = jnp.zeros((16,), jnp.int32)
        pb_idx = base + lane
        new_mxs = []
        new_ids = []
        for j in range(_NO):
            ox1 = ox1s[j]
            oy1 = oy1s[j]
            ox2 = ox2s[j]
            oy2 = oy2s[j]
            wdt = jnp.maximum(jnp.minimum(p2, ox2) - jnp.maximum(p1, ox1),
                              0.0)
            hgt = jnp.maximum(jnp.minimum(q2, oy2) - jnp.maximum(q1, oy1),
                              0.0)
            inter = wdt * hgt
            area_o = (ox2 - ox1) * (oy2 - oy1)
            iou = inter / (area_o + area_p - inter)
            upd = iou > bov
            boj = jnp.where(upd, j, boj)
            bov = jnp.where(upd, iou, bov)
            upd2 = iou > mxs[j]
            new_ids.append(jnp.where(upd2, pb_idx, ids[j]))
            new_mxs.append(jnp.where(upd2, iou, mxs[j]))
        ovb[pl.ds(base, 16)] = bov
        ojb[pl.ds(base, 16)] = boj
        return tuple(new_mxs), tuple(new_ids)

    mx0 = tuple(jnp.full((16,), -1.0, jnp.float32) for _ in range(_NO))
    id0 = tuple(jnp.zeros((16,), jnp.int32) for _ in range(_NO))
    mxs, ids = lax.fori_loop(0, _NCH, chunk, (mx0, id0))

    dn = lax.GatherDimensionNumbers(offset_dims=(),
                                    collapsed_slice_dims=(0,),
                                    start_index_map=(0,))

    def perm(x, pm):
        return lax.gather(x, pm[:, None], dn, (1,),
                          mode=lax.GatherScatterMode.PROMISE_IN_BOUNDS)

    mxs_l = list(mxs)
    ids_l = list(ids)
    for r in (8, 4, 2, 1):
        pm = (lane + r) & 15
        for j in range(_NO):
            my = perm(mxs_l[j], pm)
            iy = perm(ids_l[j], pm)
            take = (my > mxs_l[j]) | ((my == mxs_l[j]) & (iy < ids_l[j]))
            ids_l[j] = jnp.where(take, iy, ids_l[j])
            mxs_l[j] = jnp.maximum(mxs_l[j], my)

    def selchunk(c, carry):
        base = c * 16
        ovv = ovb[pl.ds(base, 16)]
        ojv = ojb[pl.ds(base, 16)]
        sel = ojv + jnp.where(ovv < 0.5, 16, 0)
        pb_idx = base + lane
        for j in range(_NO):
            sel = jnp.where(pb_idx == ids_l[j], j, sel)
        selb[pl.ds(base, 16)] = sel
        return carry

    lax.fori_loop(0, _NCH, selchunk, 0)
    pltpu.sync_copy(selb, out_hbm.at[i])


def _sc_match(obj4, pxy):
    f = pl.kernel(
        _sc_match_body,
        mesh=plsc.VectorSubcoreMesh(core_axis_name="c", subcore_axis_name="s"),
        out_type=jax.ShapeDtypeStruct((_BS, _NPP), jnp.int32),
        scratch_types=[
            pltpu.VMEM((4, 16), jnp.float32),
            pltpu.VMEM((_NPP,), jnp.float32),
            pltpu.VMEM((_NPP,), jnp.float32),
            pltpu.VMEM((_NPP,), jnp.float32),
            pltpu.VMEM((_NPP,), jnp.float32),
            pltpu.VMEM((_NPP,), jnp.float32),
            pltpu.VMEM((_NPP,), jnp.int32),
            pltpu.VMEM((_NPP,), jnp.int32),
        ],
    )
    return f(obj4, pxy)


def _image_body(sel_ref, tb_ref, tc_ref, pbc_ref, pb_ref, sc_ref,
                cls_ref, stats_ref):
    selp = sel_ref[0][:, :_NP]
    neg = selp >= 16
    obj = jnp.where(neg, selp - 16, selp)

    jidx = lax.broadcasted_iota(jnp.int32, (_NO, _NP), 0)
    onehotf = (obj == jidx).astype(jnp.float32)
    tb = tb_ref[0]
    tb5 = jnp.concatenate([tb, tc_ref[0].astype(jnp.float32)], axis=1)
    gath = lax.dot_general(tb5, onehotf, (((0,), (0,)), ((), ())),
                           preferred_element_type=jnp.float32)
    gx1 = gath[0:1, :]
    gy1 = gath[1:2, :]
    gx2 = gath[2:3, :]
    gy2 = gath[3:4, :]
    labels = jnp.where(neg, 0, gath[4:5, :].astype(jnp.int32))

    cx = (gx1 + gx2) * 0.5
    cy = (gy1 + gy2) * 0.5
    bw = gx2 - gx1
    bh = gy2 - gy1
    pcx = pbc_ref[0:1, :]
    pcy = pbc_ref[1:2, :]
    pw = pbc_ref[2:3, :]
    ph = pbc_ref[3:4, :]
    gcx = (cx - pcx) / (pw * 0.1)
    gcy = (cy - pcy) / (ph * 0.1)
    gw = jnp.log(bw / pw) * 5.0
    gh = jnp.log(bh / ph) * 5.0

    posf = (labels != 0).astype(jnp.float32)
    pb = pb_ref[0]
    locsum = (jnp.sum(jnp.abs(pb[0:1, :] - gcx) * posf)
              + jnp.sum(jnp.abs(pb[1:2, :] - gcy) * posf)
              + jnp.sum(jnp.abs(pb[2:3, :] - gw) * posf)
              + jnp.sum(jnp.abs(pb[3:4, :] - gh) * posf))
    npos = jnp.sum(posf)

    s = sc_ref[0]
    m = jnp.max(s)
    e = jnp.exp(s - m)
    ones = jnp.ones((1, _NC), jnp.float32)
    se_t = lax.dot_general(ones, e, (((1,), (1,)), ((), ())),
                           preferred_element_type=jnp.float32)
    lse_t = jnp.log(se_t) + m

    tcc = jnp.concatenate([tc_ref[0], jnp.zeros((1, 1), jnp.int32)],
                          axis=0)
    cid81 = lax.broadcasted_iota(jnp.int32, (_NO + 1, _NC), 1)
    tco = (cid81 == tcc).astype(jnp.float32)
    cand = lax.dot_general(tco, s, (((1,), (1,)), ((), ())),
                           preferred_element_type=jnp.float32)
    k = jnp.where(neg, _NO, obj)
    rid = lax.broadcasted_iota(jnp.int32, (_NO + 1, _NP), 0)
    csel = jnp.where(rid == k, cand, 0.0)
    ones17 = jnp.ones((1, _NO + 1), jnp.float32)
    s_at_t = lax.dot_general(ones17, csel, (((1,), (0,)), ((), ())),
                             preferred_element_type=jnp.float32)
    cls_t = lse_t - s_at_t
    cls_ref[0] = cls_t

    stats_ref[0, 0:1, :] = jnp.full((1, 128), locsum, jnp.float32)
    stats_ref[0, 1:2, :] = jnp.full((1, 128), npos, jnp.float32)
    stats_ref[0, 2:3, :] = jnp.full((1, 128), jnp.sum(cls_t * posf),
                                    jnp.float32)
    stats_ref[0, 3:4, :] = jnp.zeros((1, 128), jnp.float32)


def _combine_body(cls_ref, stats_ref, out_ref):
    cls = cls_ref[...]
    stats = stats_ref[...]
    locsum = jnp.sum(stats[:, 0:1, 0:1])
    npos = jnp.sum(stats[:, 1:2, 0:1])
    clspos = jnp.sum(stats[:, 2:3, 0:1])

    k = jnp.minimum((3.0 * npos).astype(jnp.int32), _NP)
    bits = lax.bitcast_convert_type(cls, jnp.int32)

    def step(_, carry):
        lo, hi = carry
        mid = lo + ((hi - lo) >> 1)
        cnt = jnp.sum((bits >= mid).astype(jnp.int32), axis=2,
                      keepdims=True)
        ge = cnt >= k
        return jnp.where(ge, mid, lo), jnp.where(ge, hi, mid)

    lo0 = jnp.zeros((_BS, 1, 1), jnp.int32)
    hi0 = jnp.full((_BS, 1, 1), 0x7F800000, jnp.int32)
    lo, _ = lax.fori_loop(0, 31, step, (lo0, hi0))
    tval = lax.bitcast_convert_type(lo, jnp.float32)
    gt = bits > lo
    cnt_gt = jnp.sum(gt.astype(jnp.float32), axis=2, keepdims=True)
    sum_gt = jnp.sum(jnp.where(gt, cls, 0.0), axis=2, keepdims=True)
    topk = jnp.sum(sum_gt + (k.astype(jnp.float32) - cnt_gt) * tval)

    loss = locsum / (npos * 4.0) + (clspos + topk) / npos
    out_ref[...] = jnp.full((1, 1), loss, jnp.float32)


@jax.jit
def kernel(pred_boxes, pred_scores, true_boxes, true_classes, pboxes):
    f32 = jnp.float32
    pbc_t = pboxes.T
    pbx_t = jnp.concatenate([pbc_t[:2] - pbc_t[2:] / 2.0,
                             pbc_t[:2] + pbc_t[2:] / 2.0], axis=0)
    pad = jnp.full((4, _NPP - _NP), 2.0, f32)
    pxy = jnp.concatenate([pbx_t, pad], axis=1)
    obj4 = jnp.transpose(true_boxes, (0, 2, 1))
    tc3 = true_classes.reshape(_BS, _NO, 1).astype(jnp.int32)
    pb_t = jnp.transpose(pred_boxes, (0, 2, 1))

    sel = _sc_match(obj4, pxy)
    sel3 = sel.reshape(_BS, 1, _NPP)

    cls_all, stats = pl.pallas_call(
        _image_body,
        grid=(_BS,),
        in_specs=[
            pl.BlockSpec((1, 1, _NPP), lambda i: (i, 0, 0)),
            pl.BlockSpec((1, _NO, 4), lambda i: (i, 0, 0)),
            pl.BlockSpec((1, _NO, 1), lambda i: (i, 0, 0)),
            pl.BlockSpec((4, _NP), lambda i: (0, 0)),
            pl.BlockSpec((1, 4, _NP), lambda i: (i, 0, 0)),
            pl.BlockSpec((1, _NP, _NC), lambda i: (i, 0, 0)),
        ],
        out_specs=[
            pl.BlockSpec((1, 1, _NP), lambda i: (i, 0, 0)),
            pl.BlockSpec((1, 4, 128), lambda i: (i, 0, 0)),
        ],
        out_shape=[
            jax.ShapeDtypeStruct((_BS, 1, _NP), f32),
            jax.ShapeDtypeStruct((_BS, 4, 128), f32),
        ],
        compiler_params=pltpu.CompilerParams(
            vmem_limit_bytes=100 * 1024 * 1024),
    )(sel3, true_boxes, tc3, pbc_t, pb_t, pred_scores)

    out = pl.pallas_call(
        _combine_body,
        out_shape=jax.ShapeDtypeStruct((1, 1), f32),
    )(cls_all, stats)
    return out[0, 0]
